# baseline (speedup 1.0000x reference)
import functools
import math

import jax
import jax.numpy as jnp
from jax import lax
from jax.experimental import pallas as pl
from jax.experimental.pallas import tpu as pltpu
from jax.experimental.pallas import tpu_sc as plsc

N_NODES = 10000
N_EDGES = 320000
D = 128
N_GRAPHS = 64

NC = 2
NS = 16
NW = NC * NS
B = 128
NB = 4
C = -(-N_EDGES // (NW * B * NB)) * NB
E_PAD = NW * B * C
N_PAD = 10240
STRIPE = N_PAD // NS
ZR = 128
ROWS_BLK = 1280

_mesh = plsc.VectorSubcoreMesh(core_axis_name="c", subcore_axis_name="s")


def _sc_layer_body(p_hbm, packed_hbm, zeros_hbm, out_hbm,
                   s_sp, slab_v, rows0, rows1, src0, src1, dst0, dst1,
                   gsem0, gsem1):
    rows = (rows0, rows1)
    srcb = (src0, src1)
    dstb = (dst0, dst1)
    gsems = (gsem0, gsem1)
    cid = lax.axis_index("c")
    sid = lax.axis_index("s")
    wid = cid * NS + sid

    pltpu.sync_copy(zeros_hbm, rows0)
    for z in range(STRIPE // ZR):
        pltpu.sync_copy(rows0, s_sp.at[pl.ds(sid * STRIPE + z * ZR, ZR)])

    pltpu.sync_copy(packed_hbm.at[wid], slab_v)

    def unpack(j, sb, db):
        for k in range(B // 16):
            v = slab_v[pl.ds(j * B + k * 16, 16)]
            sb[pl.ds(k * 16, 16)] = v & jnp.int32(16383)
            db[pl.ds(k * 16, 16)] = v >> jnp.int32(14)

    unpack(0, src0, dst0)
    plsc.subcore_barrier()

    def ebody(g, _):
        for b in range(2):
            o = 1 - b
            j = g * 2 + b
            pltpu.async_copy(p_hbm.at[srcb[b]], rows[b], gsems[b]).wait()

            @pl.when(j + 1 < C)
            def _():
                unpack(j + 1, srcb[o], dstb[o])

            pltpu.sync_copy(rows[b], s_sp.at[dstb[b]], add=True)
        return 0

    lax.fori_loop(0, C // 2, ebody, 0)
    plsc.subcore_barrier()

    for z in range(STRIPE // ZR):
        r = sid * STRIPE + z * ZR
        pltpu.sync_copy(s_sp.at[pl.ds(r, ZR)], rows0)
        pltpu.sync_copy(rows0, out_hbm.at[cid, pl.ds(r, ZR)])


_sc_layer = pl.kernel(
    _sc_layer_body,
    out_type=jax.ShapeDtypeStruct((NC, N_PAD, D), jnp.float32),
    mesh=_mesh,
    scratch_types=[
        pltpu.VMEM_SHARED((N_PAD, D), jnp.float32),
        pltpu.VMEM((C * B,), jnp.int32),
        pltpu.VMEM((ZR, D), jnp.float32),
        pltpu.VMEM((ZR, D), jnp.float32),
        pltpu.VMEM((B,), jnp.int32),
        pltpu.VMEM((B,), jnp.int32),
        pltpu.VMEM((B,), jnp.int32),
        pltpu.VMEM((B,), jnp.int32),
    ] + [pltpu.SemaphoreType.DMA] * 2,
)


def _sc_scalar_body(vals_hbm, srcs_hbm, dsts_hbm, out_hbm,
                    src_v, dst_v, sval_v, buf_v, s_sp, *gsems):
    cid = lax.axis_index("c")
    sid = lax.axis_index("s")
    wid = cid * NS + sid

    zv = jnp.zeros((16,), jnp.float32)
    for k in range(STRIPE // 16):
        buf_v[pl.ds(k * 16, 16)] = zv
    pltpu.sync_copy(buf_v, s_sp.at[pl.ds(sid * STRIPE, STRIPE)])

    pltpu.sync_copy(srcs_hbm.at[wid], src_v)
    pltpu.sync_copy(dsts_hbm.at[wid], dst_v)
    for b in range(NB):
        pltpu.async_copy(vals_hbm.at[src_v.at[b]], sval_v.at[b], gsems[b])
    plsc.subcore_barrier()

    def ebody(g, _):
        for b in range(NB):
            j = g * NB + b
            pltpu.make_async_copy(vals_hbm.at[src_v.at[j]], sval_v.at[b],
                                  gsems[b]).wait()
            pltpu.sync_copy(sval_v.at[b], s_sp.at[dst_v.at[j]], add=True)

            @pl.when(j + NB < C)
            def _():
                pltpu.async_copy(vals_hbm.at[src_v.at[j + NB]], sval_v.at[b],
                                 gsems[b])
        return 0

    lax.fori_loop(0, C // NB, ebody, 0)
    plsc.subcore_barrier()

    pltpu.sync_copy(s_sp.at[pl.ds(sid * STRIPE, STRIPE)], buf_v)
    pltpu.sync_copy(buf_v, out_hbm.at[cid, pl.ds(sid * STRIPE, STRIPE)])


_sc_scalar = pl.kernel(
    _sc_scalar_body,
    out_type=jax.ShapeDtypeStruct((NC, N_PAD), jnp.float32),
    mesh=_mesh,
    scratch_types=[
        pltpu.VMEM((C, B), jnp.int32),
        pltpu.VMEM((C, B), jnp.int32),
        pltpu.VMEM((NB, B), jnp.float32),
        pltpu.VMEM((STRIPE,), jnp.float32),
        pltpu.VMEM_SHARED((N_PAD,), jnp.float32),
    ] + [pltpu.SemaphoreType.DMA] * NB,
)


def _sc_deg_body(dsts_hbm, out_hbm, dst_v, ones_v, buf_v, s_sp):
    cid = lax.axis_index("c")
    sid = lax.axis_index("s")
    wid = cid * NS + sid

    zv = jnp.zeros((16,), jnp.float32)
    for k in range(STRIPE // 16):
        buf_v[pl.ds(k * 16, 16)] = zv
    pltpu.sync_copy(buf_v, s_sp.at[pl.ds(sid * STRIPE, STRIPE)])
    ov = jnp.ones((16,), jnp.float32)
    for k in range(B // 16):
        ones_v[pl.ds(k * 16, 16)] = ov

    pltpu.sync_copy(dsts_hbm.at[wid], dst_v)
    plsc.subcore_barrier()

    def ebody(j, _):
        pltpu.sync_copy(ones_v, s_sp.at[dst_v.at[j]], add=True)
        return 0

    lax.fori_loop(0, C, ebody, 0)
    plsc.subcore_barrier()

    pltpu.sync_copy(s_sp.at[pl.ds(sid * STRIPE, STRIPE)], buf_v)
    pltpu.sync_copy(buf_v, out_hbm.at[cid, pl.ds(sid * STRIPE, STRIPE)])


_sc_deg = pl.kernel(
    _sc_deg_body,
    out_type=jax.ShapeDtypeStruct((NC, N_PAD), jnp.float32),
    mesh=_mesh,
    scratch_types=[
        pltpu.VMEM((C, B), jnp.int32),
        pltpu.VMEM((B,), jnp.float32),
        pltpu.VMEM((STRIPE,), jnp.float32),
        pltpu.VMEM_SHARED((N_PAD,), jnp.float32),
    ],
)


def _tc_first_body(deg_ref, x_ref, w_ref, dinv_ref, p_ref):
    d = deg_ref[0] + deg_ref[1] + 1.0
    dinv = lax.rsqrt(d)
    dinv_ref[...] = dinv
    h = jnp.dot(x_ref[...], w_ref[...], preferred_element_type=jnp.float32)
    p_ref[...] = dinv * h


def _tc_mid_body(s_ref, p_ref, dinv_ref, b_ref, w_ref, out_ref):
    dinv = dinv_ref[...]
    h = dinv * (s_ref[0] + s_ref[1] + p_ref[...]) + b_ref[...]
    h = jnp.maximum(h, 0.0)
    out_ref[...] = dinv * jnp.dot(h, w_ref[...],
                                  preferred_element_type=jnp.float32)


def _tc_last_body(s_ref, p_ref, dinv_ref, b_ref, w3_ref, lw_ref, out_ref):
    dinv = dinv_ref[...]
    h = dinv * (s_ref[0] + s_ref[1] + p_ref[...]) + b_ref[...]
    h = jnp.maximum(h, 0.0)
    hw = jnp.dot(h, w3_ref[...], preferred_element_type=jnp.float32)
    out_ref[...] = dinv * jnp.dot(hw, lw_ref[...],
                                  preferred_element_type=jnp.float32)


def _tc_pool_body(s3_ref, s_ref, dinv_ref, batch_ref, c3_ref, linb_ref,
                  out_ref, acc_sum, acc_cnt):
    i = pl.program_id(0)
    t = dinv_ref[...] * (s3_ref[0] + s3_ref[1] + s_ref[...]) + c3_ref[0, 0]
    row = (jax.lax.broadcasted_iota(jnp.int32, (ROWS_BLK, 1), 0)
           + i * ROWS_BLK)
    valid = row < N_NODES
    t = jnp.where(valid, t, 0.0)
    gid = jax.lax.broadcasted_iota(jnp.int32, (1, N_GRAPHS), 1)
    onehot = (batch_ref[...] == gid) & valid
    sums = jnp.sum(jnp.where(onehot, t, 0.0), axis=0, keepdims=True)
    cnts = jnp.sum(jnp.where(onehot, 1.0, 0.0), axis=0, keepdims=True)

    @pl.when(i == 0)
    def _():
        acc_sum[...] = jnp.zeros_like(acc_sum)
        acc_cnt[...] = jnp.zeros_like(acc_cnt)

    acc_sum[...] += sums
    acc_cnt[...] += cnts

    @pl.when(i == pl.num_programs(0) - 1)
    def _():
        out_ref[...] = (acc_sum[...] / jnp.maximum(acc_cnt[...], 1.0)
                        + linb_ref[0, 0])


def _row_spec(width):
    return pl.BlockSpec((ROWS_BLK, width), lambda i: (i, 0))


def _pair_spec(width):
    return pl.BlockSpec((NC, ROWS_BLK, width), lambda i: (0, i, 0))


def _full_spec(shape):
    return pl.BlockSpec(shape, lambda i: tuple(0 for _ in shape))


_GRID = N_PAD // ROWS_BLK

_tc_first = pl.pallas_call(
    _tc_first_body,
    grid=(_GRID,),
    in_specs=[_pair_spec(1), _row_spec(D), _full_spec((D, D))],
    out_specs=(_row_spec(1), _row_spec(D)),
    out_shape=(jax.ShapeDtypeStruct((N_PAD, 1), jnp.float32),
               jax.ShapeDtypeStruct((N_PAD, D), jnp.float32)),
)

_tc_mid = pl.pallas_call(
    _tc_mid_body,
    grid=(_GRID,),
    in_specs=[_pair_spec(D), _row_spec(D), _row_spec(1),
              _full_spec((1, D)), _full_spec((D, D))],
    out_specs=_row_spec(D),
    out_shape=jax.ShapeDtypeStruct((N_PAD, D), jnp.float32),
)

_tc_last = pl.pallas_call(
    _tc_last_body,
    grid=(_GRID,),
    in_specs=[_pair_spec(D), _row_spec(D), _row_spec(1),
              _full_spec((1, D)), _full_spec((D, D)), _full_spec((D, 1))],
    out_specs=_row_spec(1),
    out_shape=jax.ShapeDtypeStruct((N_PAD, 1), jnp.float32),
)

_tc_pool = pl.pallas_call(
    _tc_pool_body,
    grid=(_GRID,),
    in_specs=[_pair_spec(1), _row_spec(1), _row_spec(1), _row_spec(1),
              _full_spec((1, 1)), _full_spec((1, 1))],
    out_specs=_full_spec((1, N_GRAPHS)),
    out_shape=jax.ShapeDtypeStruct((1, N_GRAPHS), jnp.float32),
    scratch_shapes=[pltpu.VMEM((1, N_GRAPHS), jnp.float32),
                    pltpu.VMEM((1, N_GRAPHS), jnp.float32)],
)


@jax.jit
def kernel(x, edge_index, batch, W1, b1, W2, b2, W3, b3, lin_W, lin_b):
    f32 = jnp.float32
    src = edge_index[0].astype(jnp.int32)
    dst = edge_index[1].astype(jnp.int32)
    pad = jnp.full((E_PAD - N_EDGES,), N_NODES, jnp.int32)
    srcs = jnp.concatenate([src, pad]).reshape(NW, C, B)
    dsts = jnp.concatenate([dst, pad]).reshape(NW, C, B)

    x_pad = jnp.zeros((N_PAD, D), f32).at[:N_NODES].set(x)
    zeros_blk = jnp.zeros((ZR, D), f32)

    deg2 = _sc_deg(dsts)
    dinv, p1 = _tc_first(deg2[..., None], x_pad, W1)

    packed = (srcs + dsts * 16384).reshape(NW, C * B)

    s1 = _sc_layer(p1, packed, zeros_blk)
    p2 = _tc_mid(s1, p1, dinv, b1[None, :], W2)

    s2 = _sc_layer(p2, packed, zeros_blk)
    sval = _tc_last(s2, p2, dinv, b2[None, :], W3, lin_W)

    s3 = _sc_scalar(sval[:, 0], srcs, dsts)
    c3 = jnp.dot(b3, lin_W).reshape(1, 1)
    batch_pad = jnp.zeros((N_PAD, 1), jnp.int32).at[:N_NODES, 0].set(batch)
    g = _tc_pool(s3[..., None], sval, dinv, batch_pad, c3,
                 lin_b.reshape(1, 1))
    return g.reshape(N_GRAPHS, 1)


# speedup vs baseline: 1.9430x; 1.9430x over previous
import functools
import math

import jax
import jax.numpy as jnp
from jax import lax
from jax.experimental import pallas as pl
from jax.experimental.pallas import tpu as pltpu
from jax.experimental.pallas import tpu_sc as plsc

N_NODES = 10000
N_EDGES = 320000
D = 128
N_GRAPHS = 64

NC = 2
NS = 16
NW = NC * NS
B = 128
NB = 4
C = -(-N_EDGES // (NW * B * NB)) * NB
E_PAD = NW * B * C
N_PAD = 10240
STRIPE = N_PAD // NS
ZR = 128
ZR2 = 32
ROWS_BLK = 1280

_mesh = plsc.VectorSubcoreMesh(core_axis_name="c", subcore_axis_name="s")


CF = 102
CS = 56
E_PAD_L = NS * (CF + CS) * B


def _sc_layer_body(p_hbm, srcs_hbm, dsts_hbm, zeros_hbm, out_hbm,
                   src_v, dst_v, rows_v, buf_v, s_sp, gsem):
    cid = lax.axis_index("c")
    sid = lax.axis_index("s")
    wid = cid * NS + sid
    cn = jnp.where(cid == 0, CF, CS)

    pltpu.sync_copy(zeros_hbm, buf_v)
    for z in range(STRIPE // ZR2):
        pltpu.sync_copy(buf_v, s_sp.at[pl.ds(sid * STRIPE + z * ZR2, ZR2)])

    pltpu.sync_copy(srcs_hbm.at[wid], src_v)
    pltpu.sync_copy(dsts_hbm.at[wid], dst_v)
    plsc.subcore_barrier()

    def ebody(j, _):
        pltpu.async_copy(p_hbm.at[src_v.at[j]], rows_v, gsem).wait()
        pltpu.sync_copy(rows_v, s_sp.at[dst_v.at[j]], add=True)
        return 0

    lax.fori_loop(0, cn, ebody, 0)
    plsc.subcore_barrier()

    for z in range(STRIPE // ZR2):
        r = sid * STRIPE + z * ZR2
        pltpu.sync_copy(s_sp.at[pl.ds(r, ZR2)], buf_v)
        pltpu.sync_copy(buf_v, out_hbm.at[cid, pl.ds(r, ZR2)])


_sc_layer = pl.kernel(
    _sc_layer_body,
    out_type=jax.ShapeDtypeStruct((NC, N_PAD, D), jnp.float32),
    mesh=_mesh,
    scratch_types=[
        pltpu.VMEM((CF, B), jnp.int32),
        pltpu.VMEM((CF, B), jnp.int32),
        pltpu.VMEM((B, D), jnp.float32),
        pltpu.VMEM((ZR2, D), jnp.float32),
        pltpu.VMEM_SHARED((N_PAD, D), jnp.float32),
        pltpu.SemaphoreType.DMA,
    ],
)


def _sc_scalar_body(vals_hbm, srcs_hbm, dsts_hbm, out_hbm,
                    src_v, dst_v, sval_v, buf_v, s_sp, *gsems):
    cid = lax.axis_index("c")
    sid = lax.axis_index("s")
    wid = cid * NS + sid

    zv = jnp.zeros((16,), jnp.float32)
    for k in range(STRIPE // 16):
        buf_v[pl.ds(k * 16, 16)] = zv
    pltpu.sync_copy(buf_v, s_sp.at[pl.ds(sid * STRIPE, STRIPE)])

    pltpu.sync_copy(srcs_hbm.at[wid], src_v)
    pltpu.sync_copy(dsts_hbm.at[wid], dst_v)
    for b in range(NB):
        pltpu.async_copy(vals_hbm.at[src_v.at[b]], sval_v.at[b], gsems[b])
    plsc.subcore_barrier()

    def ebody(g, _):
        for b in range(NB):
            j = g * NB + b
            pltpu.make_async_copy(vals_hbm.at[src_v.at[j]], sval_v.at[b],
                                  gsems[b]).wait()
            pltpu.sync_copy(sval_v.at[b], s_sp.at[dst_v.at[j]], add=True)

            @pl.when(j + NB < C)
            def _():
                pltpu.async_copy(vals_hbm.at[src_v.at[j + NB]], sval_v.at[b],
                                 gsems[b])
        return 0

    lax.fori_loop(0, C // NB, ebody, 0)
    plsc.subcore_barrier()

    pltpu.sync_copy(s_sp.at[pl.ds(sid * STRIPE, STRIPE)], buf_v)
    pltpu.sync_copy(buf_v, out_hbm.at[cid, pl.ds(sid * STRIPE, STRIPE)])


_sc_scalar = pl.kernel(
    _sc_scalar_body,
    out_type=jax.ShapeDtypeStruct((NC, N_PAD), jnp.float32),
    mesh=_mesh,
    scratch_types=[
        pltpu.VMEM((C, B), jnp.int32),
        pltpu.VMEM((C, B), jnp.int32),
        pltpu.VMEM((NB, B), jnp.float32),
        pltpu.VMEM((STRIPE,), jnp.float32),
        pltpu.VMEM_SHARED((N_PAD,), jnp.float32),
    ] + [pltpu.SemaphoreType.DMA] * NB,
)


def _sc_deg_body(dsts_hbm, out_hbm, dst_v, ones_v, buf_v, s_sp):
    cid = lax.axis_index("c")
    sid = lax.axis_index("s")
    wid = cid * NS + sid

    zv = jnp.zeros((16,), jnp.float32)
    for k in range(STRIPE // 16):
        buf_v[pl.ds(k * 16, 16)] = zv
    pltpu.sync_copy(buf_v, s_sp.at[pl.ds(sid * STRIPE, STRIPE)])
    ov = jnp.ones((16,), jnp.float32)
    for k in range(B // 16):
        ones_v[pl.ds(k * 16, 16)] = ov

    pltpu.sync_copy(dsts_hbm.at[wid], dst_v)
    plsc.subcore_barrier()

    def ebody(j, _):
        pltpu.sync_copy(ones_v, s_sp.at[dst_v.at[j]], add=True)
        return 0

    lax.fori_loop(0, C, ebody, 0)
    plsc.subcore_barrier()

    pltpu.sync_copy(s_sp.at[pl.ds(sid * STRIPE, STRIPE)], buf_v)
    pltpu.sync_copy(buf_v, out_hbm.at[cid, pl.ds(sid * STRIPE, STRIPE)])


_sc_deg = pl.kernel(
    _sc_deg_body,
    out_type=jax.ShapeDtypeStruct((NC, N_PAD), jnp.float32),
    mesh=_mesh,
    scratch_types=[
        pltpu.VMEM((C, B), jnp.int32),
        pltpu.VMEM((B,), jnp.float32),
        pltpu.VMEM((STRIPE,), jnp.float32),
        pltpu.VMEM_SHARED((N_PAD,), jnp.float32),
    ],
)


def _tc_first_body(deg_ref, x_ref, w_ref, dinv_ref, p_ref):
    d = deg_ref[0] + deg_ref[1] + 1.0
    dinv = lax.rsqrt(d)
    dinv_ref[...] = dinv
    h = jnp.dot(x_ref[...], w_ref[...], preferred_element_type=jnp.float32)
    p_ref[...] = dinv * h


def _tc_mid_body(s_ref, p_ref, dinv_ref, b_ref, w_ref, out_ref):
    dinv = dinv_ref[...]
    h = dinv * (s_ref[0] + s_ref[1] + p_ref[...]) + b_ref[...]
    h = jnp.maximum(h, 0.0)
    out_ref[...] = dinv * jnp.dot(h, w_ref[...],
                                  preferred_element_type=jnp.float32)


def _tc_last_body(s_ref, p_ref, dinv_ref, b_ref, w3_ref, lw_ref, out_ref):
    dinv = dinv_ref[...]
    h = dinv * (s_ref[0] + s_ref[1] + p_ref[...]) + b_ref[...]
    h = jnp.maximum(h, 0.0)
    hw = jnp.dot(h, w3_ref[...], preferred_element_type=jnp.float32)
    out_ref[...] = dinv * jnp.dot(hw, lw_ref[...],
                                  preferred_element_type=jnp.float32)


def _tc_pool_body(s3_ref, s_ref, dinv_ref, batch_ref, c3_ref, linb_ref,
                  out_ref, acc_sum, acc_cnt):
    i = pl.program_id(0)
    t = dinv_ref[...] * (s3_ref[0] + s3_ref[1] + s_ref[...]) + c3_ref[0, 0]
    row = (jax.lax.broadcasted_iota(jnp.int32, (ROWS_BLK, 1), 0)
           + i * ROWS_BLK)
    valid = row < N_NODES
    t = jnp.where(valid, t, 0.0)
    gid = jax.lax.broadcasted_iota(jnp.int32, (1, N_GRAPHS), 1)
    onehot = (batch_ref[...] == gid) & valid
    sums = jnp.sum(jnp.where(onehot, t, 0.0), axis=0, keepdims=True)
    cnts = jnp.sum(jnp.where(onehot, 1.0, 0.0), axis=0, keepdims=True)

    @pl.when(i == 0)
    def _():
        acc_sum[...] = jnp.zeros_like(acc_sum)
        acc_cnt[...] = jnp.zeros_like(acc_cnt)

    acc_sum[...] += sums
    acc_cnt[...] += cnts

    @pl.when(i == pl.num_programs(0) - 1)
    def _():
        out_ref[...] = (acc_sum[...] / jnp.maximum(acc_cnt[...], 1.0)
                        + linb_ref[0, 0])


def _row_spec(width):
    return pl.BlockSpec((ROWS_BLK, width), lambda i: (i, 0))


def _pair_spec(width):
    return pl.BlockSpec((NC, ROWS_BLK, width), lambda i: (0, i, 0))


def _full_spec(shape):
    return pl.BlockSpec(shape, lambda i: tuple(0 for _ in shape))


_GRID = N_PAD // ROWS_BLK

_tc_first = pl.pallas_call(
    _tc_first_body,
    grid=(_GRID,),
    in_specs=[_pair_spec(1), _row_spec(D), _full_spec((D, D))],
    out_specs=(_row_spec(1), _row_spec(D)),
    out_shape=(jax.ShapeDtypeStruct((N_PAD, 1), jnp.float32),
               jax.ShapeDtypeStruct((N_PAD, D), jnp.float32)),
)

_tc_mid = pl.pallas_call(
    _tc_mid_body,
    grid=(_GRID,),
    in_specs=[_pair_spec(D), _row_spec(D), _row_spec(1),
              _full_spec((1, D)), _full_spec((D, D))],
    out_specs=_row_spec(D),
    out_shape=jax.ShapeDtypeStruct((N_PAD, D), jnp.float32),
)

_tc_last = pl.pallas_call(
    _tc_last_body,
    grid=(_GRID,),
    in_specs=[_pair_spec(D), _row_spec(D), _row_spec(1),
              _full_spec((1, D)), _full_spec((D, D)), _full_spec((D, 1))],
    out_specs=_row_spec(1),
    out_shape=jax.ShapeDtypeStruct((N_PAD, 1), jnp.float32),
)

_tc_pool = pl.pallas_call(
    _tc_pool_body,
    grid=(_GRID,),
    in_specs=[_pair_spec(1), _row_spec(1), _row_spec(1), _row_spec(1),
              _full_spec((1, 1)), _full_spec((1, 1))],
    out_specs=_full_spec((1, N_GRAPHS)),
    out_shape=jax.ShapeDtypeStruct((1, N_GRAPHS), jnp.float32),
    scratch_shapes=[pltpu.VMEM((1, N_GRAPHS), jnp.float32),
                    pltpu.VMEM((1, N_GRAPHS), jnp.float32)],
)


@jax.jit
def kernel(x, edge_index, batch, W1, b1, W2, b2, W3, b3, lin_W, lin_b):
    f32 = jnp.float32
    src = edge_index[0].astype(jnp.int32)
    dst = edge_index[1].astype(jnp.int32)
    pad = jnp.full((E_PAD - N_EDGES,), N_NODES, jnp.int32)
    srcs = jnp.concatenate([src, pad]).reshape(NW, C, B)
    dsts = jnp.concatenate([dst, pad]).reshape(NW, C, B)

    x_pad = jnp.zeros((N_PAD, D), f32).at[:N_NODES].set(x)
    zeros_blk = jnp.zeros((ZR2, D), f32)

    deg2 = _sc_deg(dsts)
    dinv, p1 = _tc_first(deg2[..., None], x_pad, W1)

    npadl = jnp.full((E_PAD_L - N_EDGES,), N_NODES, jnp.int32)
    def asym(v):
        vp = jnp.concatenate([v, npadl])
        fast = vp[:NS * CF * B].reshape(NS, CF, B)
        slow = vp[NS * CF * B:].reshape(NS, CS, B)
        slow = jnp.concatenate(
            [slow, jnp.full((NS, CF - CS, B), N_NODES, jnp.int32)], axis=1)
        return jnp.concatenate([fast, slow], axis=0)
    asrcs = asym(src)
    adsts = asym(dst)

    s1 = _sc_layer(p1, asrcs, adsts, zeros_blk)
    p2 = _tc_mid(s1, p1, dinv, b1[None, :], W2)

    s2 = _sc_layer(p2, asrcs, adsts, zeros_blk)
    sval = _tc_last(s2, p2, dinv, b2[None, :], W3, lin_W)

    s3 = _sc_scalar(sval[:, 0], srcs, dsts)
    c3 = jnp.dot(b3, lin_W).reshape(1, 1)
    batch_pad = jnp.zeros((N_PAD, 1), jnp.int32).at[:N_NODES, 0].set(batch)
    g = _tc_pool(s3[..., None], sval, dinv, batch_pad, c3,
                 lin_b.reshape(1, 1))
    return g.reshape(N_GRAPHS, 1)
